# Initial kernel scaffold; baseline (speedup 1.0000x reference)
import jax
import jax.numpy as jnp
from jax.experimental import pallas as pl


def kernel(x1, x2, S1, S2):
    raise NotImplementedError("write your pallas kernel here")


# speedup vs baseline: 6.7622x; 6.7622x over previous
import jax
import jax.numpy as jnp
import numpy as np
from jax.experimental import pallas as pl
from jax.experimental.pallas import tpu as pltpu

_B, _C, _H, _W, _D = 16, 512, 14, 14, 8192
_NPIX = _B * _H * _W
_D1, _D2 = 64, 128
_PA = 784
_PB = 32

_CompilerParams = getattr(pltpu, "CompilerParams", None) or pltpu.TPUCompilerParams


def _trig():
    i64 = np.arange(_D1, dtype=np.float64)
    i128 = np.arange(_D2, dtype=np.float64)
    tp = 2.0 * np.pi
    f32 = np.float32
    c64f = np.cos(tp * np.outer(i64, i64) / _D1).astype(f32)
    s64fn = (-np.sin(tp * np.outer(i64, i64) / _D1)).astype(f32)
    twf = tp * np.outer(i64, i128) / _D
    twfr = np.cos(twf).astype(f32)[:, None, :]
    twfi = (-np.sin(twf)).astype(f32)[:, None, :]
    c128 = np.cos(tp * np.outer(i128, i128) / _D2).astype(f32)
    s128 = np.sin(tp * np.outer(i128, i128) / _D2).astype(f32)
    twir = np.cos(twf).astype(f32)[:, None, :]
    twii = np.sin(twf).astype(f32)[:, None, :]
    c64i = np.cos(tp * np.outer(i64, i64) / _D1).astype(f32)
    s64i = np.sin(tp * np.outer(i64, i64) / _D1).astype(f32)
    return c64f, s64fn, twfr, twfi, c128, s128, twir, twii, c64i, s64i


_TRIG = _trig()


def _proj_kernel(x_ref, s_ref, y_ref):
    xb = x_ref[0]
    for a in range(_D // 256):
        v = jnp.dot(xb, s_ref[0, :, a * 256:(a + 1) * 256],
                    preferred_element_type=jnp.float32)
        v = v.astype(jnp.bfloat16)
        y_ref[0, 2 * a] = v[:, :128]
        y_ref[0, 2 * a + 1] = v[:, 128:]


def _fft_kernel(y_ref, c64f, s64fn, twfr, twfi, c128, s128, twir, twii,
                c64i, s64i, out_ref):
    f32 = jnp.float32

    def fwd(i):
        y3 = y_ref[i].astype(f32)
        are = jnp.einsum('ka,apm->kpm', c64f[...], y3,
                         preferred_element_type=f32)
        aim = jnp.einsum('ka,apm->kpm', s64fn[...], y3,
                         preferred_element_type=f32)
        bre = are * twfr[...] - aim * twfi[...]
        bim = are * twfi[...] + aim * twfr[...]
        yre = (jnp.einsum('kpm,mn->kpn', bre, c128[...], preferred_element_type=f32)
               + jnp.einsum('kpm,mn->kpn', bim, s128[...], preferred_element_type=f32))
        yim = (jnp.einsum('kpm,mn->kpn', bim, c128[...], preferred_element_type=f32)
               - jnp.einsum('kpm,mn->kpn', bre, s128[...], preferred_element_type=f32))
        return yre, yim

    y1re, y1im = fwd(0)
    y2re, y2im = fwd(1)
    fre = y1re * y2re - y1im * y2im
    fim = y1re * y2im + y1im * y2re
    p1re = (jnp.einsum('kpm,mn->kpn', fre, c128[...], preferred_element_type=f32)
            - jnp.einsum('kpm,mn->kpn', fim, s128[...], preferred_element_type=f32))
    p1im = (jnp.einsum('kpm,mn->kpn', fre, s128[...], preferred_element_type=f32)
            + jnp.einsum('kpm,mn->kpn', fim, c128[...], preferred_element_type=f32))
    p2re = p1re * twir[...] - p1im * twii[...]
    p2im = p1re * twii[...] + p1im * twir[...]
    outv = (jnp.einsum('na,apm->npm', c64i[...], p2re, preferred_element_type=f32)
            - jnp.einsum('na,apm->npm', s64i[...], p2im, preferred_element_type=f32))
    out_ref[...] = outv


def kernel(x1, x2, S1, S2):
    bf16 = jnp.bfloat16
    xs = jnp.stack([
        x1.transpose(0, 2, 3, 1).reshape(_NPIX, _C),
        x2.transpose(0, 2, 3, 1).reshape(_NPIX, _C),
    ]).astype(bf16)
    ss = jnp.stack([S1, S2]).astype(bf16)

    y = pl.pallas_call(
        _proj_kernel,
        grid=(2, _NPIX // _PA),
        in_specs=[
            pl.BlockSpec((1, _PA, _C), lambda i, j: (i, j, 0)),
            pl.BlockSpec((1, _C, _D), lambda i, j: (i, 0, 0)),
        ],
        out_specs=pl.BlockSpec((1, _D1, _PA, _D2), lambda i, j: (i, 0, j, 0)),
        out_shape=jax.ShapeDtypeStruct((2, _D1, _NPIX, _D2), bf16),
        compiler_params=_CompilerParams(
            dimension_semantics=("parallel", "parallel"),
            vmem_limit_bytes=100 * 1024 * 1024,
        ),
    )(xs, ss)

    trig = [jnp.asarray(t) for t in _TRIG]
    const_specs = [pl.BlockSpec(t.shape, lambda j, n=t.ndim: (0,) * n)
                   for t in trig]

    outv = pl.pallas_call(
        _fft_kernel,
        grid=(_NPIX // _PB,),
        in_specs=[pl.BlockSpec((2, _D1, _PB, _D2), lambda j: (0, 0, j, 0))]
        + const_specs,
        out_specs=pl.BlockSpec((_D1, _PB, _D2), lambda j: (0, j, 0)),
        out_shape=jax.ShapeDtypeStruct((_D1, _NPIX, _D2), jnp.float32),
        compiler_params=_CompilerParams(
            dimension_semantics=("parallel",),
            vmem_limit_bytes=100 * 1024 * 1024,
        ),
    )(y, *trig)

    return outv.transpose(1, 0, 2).reshape(_B, _H, _W, _D)
